# baseline (speedup 1.0000x reference)
import functools

import jax
import jax.numpy as jnp
from jax.experimental import pallas as pl

K = 10


def _fused_kernel(x_ref, wt_ref, b_ref, tab_ref, tags_ref, sem_ref, *, classes):
    x = x_ref[...]
    logits = jnp.dot(x, wt_ref[...], preferred_element_type=jnp.float32)
    logits = logits + b_ref[...]
    m = jnp.max(logits, axis=1, keepdims=True)
    e = jnp.exp(logits - m)
    s = jnp.sum(e, axis=1, keepdims=True)
    tags = e / s
    tags_ref[...] = tags

    iota = jax.lax.broadcasted_iota(jnp.int32, tags.shape, 1)
    tab = tab_ref[...]
    work = tags
    for k in range(K):
        mx = jnp.max(work, axis=1, keepdims=True)
        cand = jnp.where(work == mx, iota, classes)
        idxk = jnp.min(cand, axis=1, keepdims=True)
        hit = iota == idxk
        onehot = hit.astype(jnp.float32)
        row = jnp.dot(onehot, tab, preferred_element_type=jnp.float32)
        sem_ref[:, k * row.shape[1]:(k + 1) * row.shape[1]] = row
        work = jnp.where(hit, -1.0, work)


def kernel(avg_features, W, b, embed_table):
    B, fc_in = avg_features.shape
    classes, sem_dim = embed_table.shape
    tile = 512
    grid = (B // tile,)

    wt = W.T
    b2 = b.reshape(1, classes)

    out_type = (
        jax.ShapeDtypeStruct((B, classes), jnp.float32),
        jax.ShapeDtypeStruct((B, K * sem_dim), jnp.float32),
    )
    tags, sem = pl.pallas_call(
        functools.partial(_fused_kernel, classes=classes),
        grid=grid,
        in_specs=[
            pl.BlockSpec((tile, fc_in), lambda i: (i, 0)),
            pl.BlockSpec((fc_in, classes), lambda i: (0, 0)),
            pl.BlockSpec((1, classes), lambda i: (0, 0)),
            pl.BlockSpec((classes, sem_dim), lambda i: (0, 0)),
        ],
        out_specs=(
            pl.BlockSpec((tile, classes), lambda i: (i, 0)),
            pl.BlockSpec((tile, K * sem_dim), lambda i: (i, 0)),
        ),
        out_shape=out_type,
    )(avg_features, wt, b2, embed_table)
    return (tags, sem.reshape(B, K, sem_dim))


# speedup vs baseline: 1.4366x; 1.4366x over previous
import functools

import jax
import jax.numpy as jnp
from jax.experimental import pallas as pl

K = 10


def _fused_kernel(x_ref, wt_ref, b_ref, tab_ref, tags_ref, sem_ref, *, classes):
    x = x_ref[...]
    logits = jnp.dot(x, wt_ref[...], preferred_element_type=jnp.float32)
    logits = logits + b_ref[...]
    m = jnp.max(logits, axis=1, keepdims=True)
    e = jnp.exp(logits - m)
    s = jnp.sum(e, axis=1, keepdims=True)
    tags = e / s
    tags_ref[...] = tags

    iota = jax.lax.broadcasted_iota(jnp.int32, tags.shape, 1)
    tab = tab_ref[...]
    work = tags
    for k in range(K):
        mx = jnp.max(work, axis=1, keepdims=True)
        cand = jnp.where(work == mx, iota, classes)
        idxk = jnp.min(cand, axis=1, keepdims=True)
        hit = iota == idxk
        onehot = hit.astype(jnp.float32)
        row = jnp.dot(onehot, tab, preferred_element_type=jnp.float32)
        sem_ref[:, k, :] = row
        work = jnp.where(hit, -1.0, work)


def kernel(avg_features, W, b, embed_table):
    B, fc_in = avg_features.shape
    classes, sem_dim = embed_table.shape
    tile = 512
    grid = (B // tile,)

    wt = W.T
    b2 = b.reshape(1, classes)

    out_type = (
        jax.ShapeDtypeStruct((B, classes), jnp.float32),
        jax.ShapeDtypeStruct((B, 16, sem_dim), jnp.float32),
    )
    tags, sem = pl.pallas_call(
        functools.partial(_fused_kernel, classes=classes),
        grid=grid,
        in_specs=[
            pl.BlockSpec((tile, fc_in), lambda i: (i, 0)),
            pl.BlockSpec((fc_in, classes), lambda i: (0, 0)),
            pl.BlockSpec((1, classes), lambda i: (0, 0)),
            pl.BlockSpec((classes, sem_dim), lambda i: (0, 0)),
        ],
        out_specs=(
            pl.BlockSpec((tile, classes), lambda i: (i, 0)),
            pl.BlockSpec((tile, 16, sem_dim), lambda i: (i, 0, 0)),
        ),
        out_shape=out_type,
    )(avg_features, wt, b2, embed_table)
    return (tags, sem[:, :K, :])
